# baseline (speedup 1.0000x reference)
import jax
import jax.numpy as jnp
from jax.experimental import pallas as pl

NUM_Q = 8
K = 1024
DIM = 64
COMMIT_W = 1.0
BLK = 1024


def _rvq_kernel(y_ref, cb_ref, yhat_ref, idx_ref, ssq_ref):
    i = pl.program_id(0)

    @pl.when(i == 0)
    def _init():
        ssq_ref[...] = jnp.zeros_like(ssq_ref)

    y = y_ref[...]
    blk = y.shape[0]
    res = y
    lane_iota = jax.lax.broadcasted_iota(jnp.int32, (blk, K), 1)
    q_iota = jax.lax.broadcasted_iota(jnp.int32, (1, NUM_Q), 1)
    idx_cols = []
    ssq_acc = jnp.zeros((1, NUM_Q), jnp.float32)
    mm = lambda a, b, dims: jax.lax.dot_general(
        a, b, (dims, ((), ())), preferred_element_type=jnp.float32)
    for qi in range(NUM_Q):
        cb = cb_ref[qi]
        s1 = cb.astype(jnp.bfloat16)
        r1 = cb - s1.astype(jnp.float32)
        s2 = r1.astype(jnp.bfloat16)
        s3 = (r1 - s2.astype(jnp.float32)).astype(jnp.bfloat16)
        x2 = jnp.sum(res * res, axis=1, keepdims=True)
        c2 = jnp.sum(cb * cb, axis=1)[None, :]
        xc = mm(res.astype(jnp.bfloat16), s1, ((1,), (1,)))
        d = x2 - 2.0 * xc + c2
        idx = jnp.argmin(d, axis=1).astype(jnp.int32)[:, None]
        onehot = (lane_iota == idx).astype(jnp.bfloat16)
        q = ((mm(onehot, s1, ((1,), (0,))) + mm(onehot, s2, ((1,), (0,))))
             + mm(onehot, s3, ((1,), (0,))))
        res = res - q
        ssq_acc = ssq_acc + jnp.where(q_iota == qi,
                                      jnp.sum(res * res), 0.0)
        idx_cols.append(idx)
    yhat_ref[...] = y - res
    idx_ref[...] = jnp.concatenate(idx_cols, axis=1)
    ssq_ref[...] += ssq_acc


def kernel(y, codebooks):
    b, _ = y.shape
    grid = (b // BLK,)
    yhat, idx, ssq = pl.pallas_call(
        _rvq_kernel,
        grid=grid,
        in_specs=[
            pl.BlockSpec((BLK, DIM), lambda i: (i, 0)),
            pl.BlockSpec((NUM_Q, K, DIM), lambda i: (0, 0, 0)),
        ],
        out_specs=[
            pl.BlockSpec((BLK, DIM), lambda i: (i, 0)),
            pl.BlockSpec((BLK, NUM_Q), lambda i: (i, 0)),
            pl.BlockSpec((1, NUM_Q), lambda i: (0, 0)),
        ],
        out_shape=[
            jax.ShapeDtypeStruct((b, DIM), jnp.float32),
            jax.ShapeDtypeStruct((b, NUM_Q), jnp.int32),
            jax.ShapeDtypeStruct((1, NUM_Q), jnp.float32),
        ],
    )(y, codebooks)
    losses_per_layer = COMMIT_W * (ssq[0] / (b * DIM))
    loss_vq = jnp.mean(losses_per_layer)
    return yhat, idx, loss_vq, losses_per_layer


# speedup vs baseline: 1.5311x; 1.5311x over previous
import jax
import jax.numpy as jnp
from jax.experimental import pallas as pl

NUM_Q = 8
K = 1024
DIM = 64
COMMIT_W = 1.0
BLK = 1024
HALF = BLK // 2


def _mm(a, b, dims):
    return jax.lax.dot_general(a, b, (dims, ((), ())),
                               preferred_element_type=jnp.float32)


def _rvq_kernel(y_ref, cb_ref, s1_ref, s2_ref, s3_ref,
                yhat_ref, idx_ref, ssq_ref):
    i = pl.program_id(0)

    @pl.when(i == 0)
    def _init():
        ssq_ref[...] = jnp.zeros_like(ssq_ref)

    lane_iota = jax.lax.broadcasted_iota(jnp.int32, (HALF, K), 1)
    q_iota = jax.lax.broadcasted_iota(jnp.int32, (1, NUM_Q), 1)
    ys = [y_ref[0:HALF, :], y_ref[HALF:BLK, :]]
    res = list(ys)
    idx_cols = [[], []]
    ssq_acc = jnp.zeros((1, NUM_Q), jnp.float32)
    for qi in range(NUM_Q):
        cb = cb_ref[qi]
        s1, s2, s3 = s1_ref[qi], s2_ref[qi], s3_ref[qi]
        c2 = jnp.sum(cb * cb, axis=1)[None, :]
        layer_ssq = 0.0
        for h in (0, 1):
            r = res[h]
            x2 = jnp.sum(r * r, axis=1, keepdims=True)
            xc = _mm(r.astype(jnp.bfloat16), s1, ((1,), (1,)))
            d = x2 - 2.0 * xc + c2
            dmin = jnp.min(d, axis=1, keepdims=True)
            idx = jnp.min(jnp.where(d == dmin, lane_iota, K),
                          axis=1, keepdims=True)
            onehot = (lane_iota == idx).astype(jnp.bfloat16)
            q = ((_mm(onehot, s1, ((1,), (0,)))
                  + _mm(onehot, s2, ((1,), (0,))))
                 + _mm(onehot, s3, ((1,), (0,))))
            r = r - q
            res[h] = r
            layer_ssq = layer_ssq + jnp.sum(r * r)
            idx_cols[h].append(idx)
        ssq_acc = ssq_acc + jnp.where(q_iota == qi, layer_ssq, 0.0)
    yhat_ref[0:HALF, :] = ys[0] - res[0]
    yhat_ref[HALF:BLK, :] = ys[1] - res[1]
    idx_ref[0:HALF, :] = jnp.concatenate(idx_cols[0], axis=1)
    idx_ref[HALF:BLK, :] = jnp.concatenate(idx_cols[1], axis=1)
    ssq_ref[...] += ssq_acc


def kernel(y, codebooks):
    b, _ = y.shape
    grid = (b // BLK,)
    s1 = codebooks.astype(jnp.bfloat16)
    r1 = codebooks - s1.astype(jnp.float32)
    s2 = r1.astype(jnp.bfloat16)
    s3 = (r1 - s2.astype(jnp.float32)).astype(jnp.bfloat16)
    cb_spec = pl.BlockSpec((NUM_Q, K, DIM), lambda i: (0, 0, 0))
    yhat, idx, ssq = pl.pallas_call(
        _rvq_kernel,
        grid=grid,
        in_specs=[pl.BlockSpec((BLK, DIM), lambda i: (i, 0)),
                  cb_spec, cb_spec, cb_spec, cb_spec],
        out_specs=[
            pl.BlockSpec((BLK, DIM), lambda i: (i, 0)),
            pl.BlockSpec((BLK, NUM_Q), lambda i: (i, 0)),
            pl.BlockSpec((1, NUM_Q), lambda i: (0, 0)),
        ],
        out_shape=[
            jax.ShapeDtypeStruct((b, DIM), jnp.float32),
            jax.ShapeDtypeStruct((b, NUM_Q), jnp.int32),
            jax.ShapeDtypeStruct((1, NUM_Q), jnp.float32),
        ],
    )(y, codebooks, s1, s2, s3)
    losses_per_layer = COMMIT_W * (ssq[0] / (b * DIM))
    loss_vq = jnp.mean(losses_per_layer)
    return yhat, idx, loss_vq, losses_per_layer
